# Initial kernel scaffold; baseline (speedup 1.0000x reference)
import jax
import jax.numpy as jnp
from jax.experimental import pallas as pl


def kernel(tensor_input, keys, values):
    raise NotImplementedError("write your pallas kernel here")


# speedup vs baseline: 3.8371x; 3.8371x over previous
import functools

import jax
import jax.numpy as jnp
from jax import lax
from jax.experimental import pallas as pl
from jax.experimental.pallas import tpu as pltpu
from jax.experimental.pallas import tpu_sc as plsc

_NROWS = 16384
_NCOLS = 16


@functools.cache
def _build():
    info = plsc.get_sparse_core_info()
    nc, ns, lanes = info.num_cores, info.num_subcores, info.num_lanes
    nw = nc * ns
    rows_per_w = _NROWS // nw
    nblk = rows_per_w // lanes

    mesh = plsc.VectorSubcoreMesh(core_axis_name="c", subcore_axis_name="s")

    @functools.partial(
        pl.kernel,
        mesh=mesh,
        out_type=jax.ShapeDtypeStruct((_NROWS,), jnp.float32),
        compiler_params=pltpu.CompilerParams(needs_layout_passes=False),
        scratch_types=[
            pltpu.VMEM((rows_per_w * _NCOLS,), jnp.float32),
            pltpu.VMEM((lanes,), jnp.int32),
            pltpu.VMEM((lanes,), jnp.float32),
            pltpu.VMEM((rows_per_w,), jnp.float32),
        ],
    )
    def sc_kernel(x_hbm, keys_hbm, vals_hbm, out_hbm, x_v, keys_v, vals_v, out_v):
        wid = lax.axis_index("s") * nc + lax.axis_index("c")
        base = wid * rows_per_w
        pltpu.sync_copy(
            x_hbm.at[pl.ds(base * _NCOLS, rows_per_w * _NCOLS)], x_v
        )
        pltpu.sync_copy(keys_hbm, keys_v)
        pltpu.sync_copy(vals_hbm, vals_v)

        lane16 = lax.iota(jnp.int32, lanes) * _NCOLS

        def body(blk, carry):
            idx0 = lane16 + blk * (lanes * _NCOLS)
            best = plsc.load_gather(x_v, [idx0])
            barg = jnp.zeros((lanes,), jnp.int32)
            for j in range(1, _NCOLS):
                vj = plsc.load_gather(x_v, [idx0 + j])
                gt = vj > best
                best = jnp.where(gt, vj, best)
                barg = jnp.where(gt, jnp.full((lanes,), j, jnp.int32), barg)
            kg = plsc.load_gather(keys_v, [barg])
            vg = plsc.load_gather(vals_v, [barg])
            res = jnp.where(kg == barg, vg, jnp.full((lanes,), -1.0, jnp.float32))
            out_v[pl.ds(blk * lanes, lanes)] = res
            return carry

        lax.fori_loop(0, nblk, body, 0)
        pltpu.sync_copy(out_v, out_hbm.at[pl.ds(base, rows_per_w)])

    return sc_kernel


def kernel(tensor_input, keys, values):
    keys32 = keys.astype(jnp.int32)
    vals32 = values.astype(jnp.float32)
    x_flat = tensor_input.reshape(-1)
    return _build()(x_flat, keys32, vals32)


# speedup vs baseline: 3.9030x; 1.0172x over previous
import functools

import jax
import jax.numpy as jnp
from jax import lax
from jax.experimental import pallas as pl
from jax.experimental.pallas import tpu as pltpu
from jax.experimental.pallas import tpu_sc as plsc

_NROWS = 16384
_NCOLS = 16


@functools.cache
def _build():
    info = plsc.get_sparse_core_info()
    nc, ns, lanes = info.num_cores, info.num_subcores, info.num_lanes
    nw = nc * ns
    rows_per_w = _NROWS // nw
    nblk = rows_per_w // lanes
    half_rows = rows_per_w // 2
    half_elems = half_rows * _NCOLS

    mesh = plsc.VectorSubcoreMesh(core_axis_name="c", subcore_axis_name="s")

    @functools.partial(
        pl.kernel,
        mesh=mesh,
        out_type=jax.ShapeDtypeStruct((_NROWS,), jnp.float32),
        compiler_params=pltpu.CompilerParams(needs_layout_passes=False),
        scratch_types=[
            pltpu.VMEM((rows_per_w * _NCOLS,), jnp.float32),
            pltpu.VMEM((lanes,), jnp.int32),
            pltpu.VMEM((lanes,), jnp.float32),
            pltpu.VMEM((rows_per_w,), jnp.float32),
            pltpu.SemaphoreType.DMA,
            pltpu.SemaphoreType.DMA,
        ],
    )
    def sc_kernel(
        x_hbm, keys_hbm, vals_hbm, out_hbm, x_v, keys_v, vals_v, out_v, sem0, sem1
    ):
        wid = lax.axis_index("s") * nc + lax.axis_index("c")
        base = wid * rows_per_w
        e0 = base * _NCOLS
        cp0 = pltpu.async_copy(
            x_hbm.at[pl.ds(e0, half_elems)], x_v.at[pl.ds(0, half_elems)], sem0
        )
        cp1 = pltpu.async_copy(
            x_hbm.at[pl.ds(e0 + half_elems, half_elems)],
            x_v.at[pl.ds(half_elems, half_elems)],
            sem1,
        )
        pltpu.sync_copy(keys_hbm, keys_v)
        pltpu.sync_copy(vals_hbm, vals_v)

        lane16 = lax.iota(jnp.int32, lanes) * _NCOLS

        def block(blk):
            idx0 = lane16 + blk * (lanes * _NCOLS)
            vals_ = [plsc.load_gather(x_v, [idx0 + j]) for j in range(_NCOLS)]
            idxs_ = [jnp.full((lanes,), j, jnp.int32) for j in range(_NCOLS)]
            while len(vals_) > 1:
                nv, ni = [], []
                for a in range(0, len(vals_), 2):
                    gt = vals_[a + 1] > vals_[a]
                    nv.append(jnp.where(gt, vals_[a + 1], vals_[a]))
                    ni.append(jnp.where(gt, idxs_[a + 1], idxs_[a]))
                vals_, idxs_ = nv, ni
            barg = idxs_[0]
            kg = plsc.load_gather(keys_v, [barg])
            vg = plsc.load_gather(vals_v, [barg])
            res = jnp.where(kg == barg, vg, jnp.full((lanes,), -1.0, jnp.float32))
            out_v[pl.ds(blk * lanes, lanes)] = res

        def body0(blk, carry):
            block(blk)
            return carry

        cp0.wait()
        lax.fori_loop(0, nblk // 2, body0, 0)
        cp1.wait()
        lax.fori_loop(nblk // 2, nblk, body0, 0)
        pltpu.sync_copy(out_v, out_hbm.at[pl.ds(base, rows_per_w)])

    return sc_kernel


def kernel(tensor_input, keys, values):
    keys32 = keys.astype(jnp.int32)
    vals32 = values.astype(jnp.float32)
    x_flat = tensor_input.reshape(-1)
    return _build()(x_flat, keys32, vals32)
